# baseline (speedup 1.0000x reference)
import jax
import jax.numpy as jnp
from jax import lax
from jax.experimental import pallas as pl
from jax.experimental.pallas import tpu as pltpu

def _assemble_kernel(w_smem, out_ref):
    row = lax.broadcasted_iota(jnp.int32, (19, 4), 0)
    col = lax.broadcasted_iota(jnp.int32, (19, 4), 1)
    acc = jnp.where(row == 5, 0.25,
                    jnp.where(row - 1 == col, 1.0, 0.0)).astype(jnp.float32)
    for i in range(13):
        for j in range(4):
            mask = jnp.logical_and(row == 6 + i, col == j)
            acc = jnp.where(mask, w_smem[i, j], acc)
    out_ref[...] = acc


def kernel(RNA_embedding_weight):
    return pl.pallas_call(
        _assemble_kernel,
        grid_spec=pltpu.PrefetchScalarGridSpec(
            num_scalar_prefetch=1,
            grid=(),
            in_specs=[],
            out_specs=pl.BlockSpec(memory_space=pltpu.VMEM),
        ),
        out_shape=jax.ShapeDtypeStruct((19, 4), jnp.float32),
    )(RNA_embedding_weight)


# speedup vs baseline: 1.0451x; 1.0451x over previous
import jax
import jax.numpy as jnp
from jax import lax
from jax.experimental import pallas as pl


def _assemble_kernel(w_ref, out_ref):
    row = lax.broadcasted_iota(jnp.int32, (6, 4), 0)
    col = lax.broadcasted_iota(jnp.int32, (6, 4), 1)
    fixed = jnp.where(row == 5, 0.25,
                      jnp.where(row - 1 == col, 1.0, 0.0)).astype(jnp.float32)
    out_ref[...] = jnp.concatenate([fixed, w_ref[...]], axis=0)


def kernel(RNA_embedding_weight):
    return pl.pallas_call(
        _assemble_kernel,
        out_shape=jax.ShapeDtypeStruct((19, 4), jnp.float32),
    )(RNA_embedding_weight)
